# Initial kernel scaffold; baseline (speedup 1.0000x reference)
import jax
import jax.numpy as jnp
from jax.experimental import pallas as pl


def kernel(x, edge_index, pe_w1, pe_b1, bn_gamma, bn_beta, bn_mean, bn_var, pe_w2, pe_b2, conv1_W, conv1_b, conv3_W, conv3_b, conv2_W, conv2_b):
    raise NotImplementedError("write your pallas kernel here")


# speedup vs baseline: 9.6805x; 9.6805x over previous
import functools

import jax
import jax.numpy as jnp
from jax import lax
from jax.experimental import pallas as pl
from jax.experimental.pallas import tpu as pltpu
from jax.experimental.pallas import tpu_sc as plsc

N = 10000
NPAD = 10240
E = 320000
NC = 2
NS = 16
NW = NC * NS
K = 128
CHUNKS = 79
EP = NW * CHUNKS * K
SLICE = NPAD // NS
R = 512
GRID = NPAD // R


def _sc_mesh():
    return plsc.VectorSubcoreMesh(
        core_axis_name="c", subcore_axis_name="s", num_cores=NC, num_subcores=NS
    )


def _deg_sc(dstc, ones_rows, z128):
    @functools.partial(
        pl.kernel,
        out_type=jax.ShapeDtypeStruct((NC, NPAD, 128), jnp.float32),
        mesh=_sc_mesh(),
        scratch_types=[
            pltpu.VMEM((CHUNKS, K), jnp.int32),
            pltpu.VMEM((K, 128), jnp.float32),
            pltpu.VMEM_SHARED((NPAD, 128), jnp.float32),
        ],
    )
    def deg_k(dst_hbm, ones_hbm, z_hbm, out_hbm, dl, onesl, acc):
        cid = lax.axis_index("c")
        sid = lax.axis_index("s")
        wid = cid * NS + sid
        pltpu.sync_copy(dst_hbm.at[wid], dl)
        pltpu.sync_copy(ones_hbm, onesl)
        pltpu.sync_copy(z_hbm.at[pl.ds(sid * SLICE, SLICE)],
                        acc.at[pl.ds(sid * SLICE, SLICE)])
        plsc.subcore_barrier()

        def body(c, carry):
            pltpu.sync_copy(onesl, acc.at[dl.at[c]], add=True)
            return carry

        lax.fori_loop(0, CHUNKS, body, 0)
        plsc.subcore_barrier()
        pltpu.sync_copy(acc.at[pl.ds(sid * SLICE, SLICE)],
                        out_hbm.at[cid, pl.ds(sid * SLICE, SLICE)])

    return deg_k(dstc, ones_rows, z128)


def _agg_sc(g, srcc, dstc, z128):
    @functools.partial(
        pl.kernel,
        out_type=jax.ShapeDtypeStruct((NC, NPAD, 128), jnp.float32),
        mesh=_sc_mesh(),
        scratch_types=[
            pltpu.VMEM((CHUNKS, K), jnp.int32),
            pltpu.VMEM((CHUNKS, K), jnp.int32),
            pltpu.VMEM((K, 128), jnp.float32),
            pltpu.VMEM_SHARED((NPAD, 128), jnp.float32),
            pltpu.SemaphoreType.DMA,
        ],
    )
    def agg_k(g_hbm, src_hbm, dst_hbm, z_hbm, out_hbm, srcl, dstl, rows, acc, sem):
        cid = lax.axis_index("c")
        sid = lax.axis_index("s")
        wid = cid * NS + sid
        pltpu.sync_copy(src_hbm.at[wid], srcl)
        pltpu.sync_copy(dst_hbm.at[wid], dstl)
        pltpu.sync_copy(z_hbm.at[pl.ds(sid * SLICE, SLICE)],
                        acc.at[pl.ds(sid * SLICE, SLICE)])
        plsc.subcore_barrier()

        def body(c, carry):
            pltpu.async_copy(g_hbm.at[srcl.at[c]], rows, sem).wait()
            pltpu.sync_copy(rows, acc.at[dstl.at[c]], add=True)
            return carry

        lax.fori_loop(0, CHUNKS, body, 0)
        plsc.subcore_barrier()
        pltpu.sync_copy(acc.at[pl.ds(sid * SLICE, SLICE)],
                        out_hbm.at[cid, pl.ds(sid * SLICE, SLICE)])

    return agg_k(g, srcc, dstc, z128)


def _row_spec(width):
    return pl.BlockSpec((R, width), lambda i: (i, 0))


def _full_spec(shape):
    return pl.BlockSpec(shape, lambda i: tuple(0 for _ in shape))


def _tc1(f, p4, dinv, A1, c1, w2T, b2, W1a, W1b):
    def body(f_ref, p4_ref, d_ref, A1_ref, c1_ref, w2T_ref, b2_ref, Wa_ref, Wb_ref, o_ref):
        ph = jnp.dot(p4_ref[...], A1_ref[...], preferred_element_type=jnp.float32)
        ph = jnp.maximum(ph + c1_ref[...], 0.0)
        pos = jnp.dot(ph, w2T_ref[...], preferred_element_type=jnp.float32) + b2_ref[...]
        h = jnp.dot(f_ref[...], Wa_ref[...], preferred_element_type=jnp.float32)
        h = h + jnp.dot(pos, Wb_ref[...], preferred_element_type=jnp.float32)
        o_ref[...] = h * d_ref[...]

    return pl.pallas_call(
        body,
        grid=(GRID,),
        in_specs=[
            _row_spec(128), _row_spec(4), _row_spec(1),
            _full_spec((4, 128)), _full_spec((1, 128)), _full_spec((128, 128)),
            _full_spec((1, 128)), _full_spec((128, 128)), _full_spec((128, 128)),
        ],
        out_specs=_row_spec(128),
        out_shape=jax.ShapeDtypeStruct((NPAD, 128), jnp.float32),
    )(f, p4, dinv, A1, c1, w2T, b2, W1a, W1b)


def _tc_mid(aggp, g, dinv, b, W, relu):
    def body(a_ref, g_ref, d_ref, b_ref, W_ref, o_ref):
        s = a_ref[0] + a_ref[1] + g_ref[...]
        out = s * d_ref[...] + b_ref[...]
        if relu:
            out = jnp.maximum(out, 0.0)
        o_ref[...] = jnp.dot(out, W_ref[...], preferred_element_type=jnp.float32) * d_ref[...]

    return pl.pallas_call(
        body,
        grid=(GRID,),
        in_specs=[
            pl.BlockSpec((2, R, 128), lambda i: (0, i, 0)),
            _row_spec(128), _row_spec(1),
            _full_spec((1, 128)), _full_spec((128, 128)),
        ],
        out_specs=_row_spec(128),
        out_shape=jax.ShapeDtypeStruct((NPAD, 128), jnp.float32),
    )(aggp, g, dinv, b, W)


def _tc_last(aggp, g, dinv, b):
    def body(a_ref, g_ref, d_ref, b_ref, o_ref):
        s = a_ref[0] + a_ref[1] + g_ref[...]
        o_ref[...] = s * d_ref[...] + b_ref[...]

    return pl.pallas_call(
        body,
        grid=(GRID,),
        in_specs=[
            pl.BlockSpec((2, R, 128), lambda i: (0, i, 0)),
            _row_spec(128), _row_spec(1),
            _full_spec((1, 128)),
        ],
        out_specs=_row_spec(128),
        out_shape=jax.ShapeDtypeStruct((NPAD, 128), jnp.float32),
    )(aggp, g, dinv, b)


def kernel(x, edge_index, pe_w1, pe_b1, bn_gamma, bn_beta, bn_mean, bn_var,
           pe_w2, pe_b2, conv1_W, conv1_b, conv3_W, conv3_b, conv2_W, conv2_b):
    f32 = jnp.float32
    x2 = x[:, 0, :]
    f = jnp.pad(x2[:, :128], ((0, NPAD - N), (0, 0)))
    p4 = jnp.pad(x2[:, 128:132], ((0, NPAD - N), (0, 0)))

    s = bn_gamma * lax.rsqrt(bn_var + 1e-5)
    A1 = pe_w1.T * s[None, :]
    c1 = ((pe_b1 - bn_mean) * s + bn_beta)[None, :]
    w2T = pe_w2.T
    b2p = pe_b2[None, :]
    W1a = conv1_W[:128]
    W1b = conv1_W[128:]

    ei = edge_index.astype(jnp.int32)
    pad_idx = jnp.full((EP - E,), N, dtype=jnp.int32)
    srcc = jnp.concatenate([ei[0], pad_idx]).reshape(NW, CHUNKS, K)
    dstc = jnp.concatenate([ei[1], pad_idx]).reshape(NW, CHUNKS, K)

    ones_rows = jnp.ones((K, 128), f32)
    z128 = jnp.zeros((NPAD, 128), f32)

    degp = _deg_sc(dstc, ones_rows, z128)
    deg = degp[0, :, 0] + degp[1, :, 0] + 1.0
    dinv = lax.rsqrt(deg)[:, None]

    g1 = _tc1(f, p4, dinv, A1, c1, w2T, b2p, W1a, W1b)
    a1 = _agg_sc(g1, srcc, dstc, z128)
    g3 = _tc_mid(a1, g1, dinv, conv1_b[None, :], conv3_W, relu=False)
    a3 = _agg_sc(g3, srcc, dstc, z128)
    g2 = _tc_mid(a3, g3, dinv, conv3_b[None, :], conv2_W, relu=True)
    a2 = _agg_sc(g2, srcc, dstc, z128)
    out = _tc_last(a2, g2, dinv, conv2_b[None, :])
    return out[:N]
